# baseline (speedup 1.0000x reference)
import functools
import math

import jax
import jax.numpy as jnp
import numpy as np
from jax import lax
from jax.experimental import pallas as pl
from jax.experimental.pallas import tpu as pltpu
from jax.experimental.pallas import tpu_sc as plsc

_NUM_BUCKETS = 32
_MAX_DISTANCE = 128
_NUM_HEADS = 16
_Q = 2048
_K = 2048
_L = 16
_ULEN = 4096
_UPAD = 4112
_R = 8


def _bucket_thresholds():
    d = np.arange(0, 4096)
    rp = d.astype(np.float32)
    tmp = np.log(rp / np.float32(16.0) + np.float32(1e-10))
    tmp = tmp / np.float32(math.log(_MAX_DISTANCE / 16))
    tmp = tmp * np.float32(16.0)
    large = np.minimum(16 + tmp.astype(np.int32), _NUM_BUCKETS - 1)
    b = np.where(d < 16, d, large)
    return [int(np.argmax(b >= k)) for k in range(1, _NUM_BUCKETS)]


_THRESHOLDS = _bucket_thresholds()


def _sc_body(tbl_hbm, out_hbm, tbl_v, u16, stage, sem):
    wid = lax.axis_index("s") * 2 + lax.axis_index("c")
    head = wid // 2
    q0 = (wid % 2) * (_Q // 2)
    row0_base = head * _Q + q0

    pltpu.sync_copy(tbl_hbm.at[head], tbl_v)

    rows = [tbl_v[b, :] for b in range(_NUM_BUCKETS)]
    v0 = rows[0]
    v31 = rows[_NUM_BUCKETS - 1]

    for s in range(_L):
        base = s * _UPAD

        def fill(lo, hi, vec, base=base):
            def body(t, carry):
                u16[pl.ds(base + t * _L, _L)] = vec
                return carry
            lax.fori_loop(lo // _L, hi // _L, body, 0)

        fill(0, 1920, v31)
        fill(2064, _ULEN, v0)

        for t in range(1920 // _L, 2064 // _L):
            jvec = jnp.arange(_L, dtype=jnp.int32) + (t * _L + s)
            dvec = 2048 - jvec
            acc = v0
            for b in range(1, _NUM_BUCKETS):
                acc = acc + jnp.where(
                    dvec >= _THRESHOLDS[b - 1], rows[b] - rows[b - 1],
                    jnp.zeros((_L,), jnp.float32),
                )
            u16[pl.ds(base + t * _L, _L)] = acc

    n_chunks = (_Q // 2) // _R

    def dma_for(c, buf):
        return pltpu.make_async_copy(
            stage.at[buf],
            out_hbm.at[pl.ds(row0_base + c * _R, _R)],
            sem,
        )

    def chunk_body(c, carry):
        buf = lax.rem(c, 2)

        @pl.when(c >= 2)
        def _wait_prev():
            dma_for(c - 2, buf).wait()

        def row_body(r, carry2):
            q = q0 + c * _R + r
            start = 2048 - q
            s = lax.rem(start, _L)
            off = s * _UPAD + (start - s)
            for t in range(_K // _L):
                stage[buf, r, pl.ds(t * _L, _L)] = u16[
                    pl.ds(off + t * _L, _L)
                ]
            return carry2

        lax.fori_loop(0, _R, row_body, 0)
        dma_for(c, buf).start()
        return carry

    lax.fori_loop(0, n_chunks, chunk_body, 0)
    dma_for(n_chunks - 2, 0).wait()
    dma_for(n_chunks - 1, 1).wait()


def kernel(query_length, key_length, relative_attention_bias):
    del query_length, key_length
    tbls = jnp.broadcast_to(
        jnp.transpose(relative_attention_bias, (1, 0))[:, :, None],
        (_NUM_HEADS, _NUM_BUCKETS, _L),
    )
    mesh = plsc.VectorSubcoreMesh(core_axis_name="c", subcore_axis_name="s")
    fn = functools.partial(
        pl.kernel,
        mesh=mesh,
        out_type=jax.ShapeDtypeStruct((_NUM_HEADS * _Q, _K), jnp.float32),
        scratch_types=[
            pltpu.VMEM((_NUM_BUCKETS, _L), jnp.float32),
            pltpu.VMEM((_L * _UPAD,), jnp.float32),
            pltpu.VMEM((2, _R, _K), jnp.float32),
            pltpu.SemaphoreType.DMA,
        ],
    )(_sc_body)
    out = fn(tbls)
    return out.reshape(_NUM_HEADS, _Q, _K)[None]
